# baseline (speedup 1.0000x reference)
import functools

import jax
import jax.numpy as jnp
from jax import lax
from jax.experimental import pallas as pl
from jax.experimental.pallas import tpu as pltpu, tpu_sc as plsc

NUM_CLS = 1000
NI, NJ = 4096, 50
NIS = 16
IW = NI // NIS
JW = NJ // 2
CW = 200
NCH = NUM_CLS // CW
CHUNKS = JW * NCH
NBUF = 2

_mesh = plsc.VectorSubcoreMesh(core_axis_name="c", subcore_axis_name="s")


@functools.partial(
    pl.kernel,
    out_type=jax.ShapeDtypeStruct((NJ, NUM_CLS, NI), jnp.float32),
    mesh=_mesh,
    scratch_types=[
        pltpu.VMEM((JW * IW,), jnp.int32),
        pltpu.VMEM((NBUF, CW, IW), jnp.float32),
        pltpu.SemaphoreType.DMA,
        pltpu.SemaphoreType.DMA,
        pltpu.SemaphoreType.DMA,
    ],
    compiler_params=pltpu.CompilerParams(needs_layout_passes=False),
)
def _onehot_sc(xt_hbm, zeros_hbm, out_hbm, idx_v, buf_v, sem0, sem1, semz):
    wid = lax.axis_index("s") * 2 + lax.axis_index("c")
    i_base = (wid % NIS) * IW
    j_base = (wid // NIS) * JW
    sems = (sem0, sem1)

    idx_cps = [
        pltpu.make_async_copy(
            xt_hbm.at[pl.ds((j_base + jj) * NI + i_base, IW)],
            idx_v.at[pl.ds(jj * IW, IW)], semz)
        for jj in range(JW)
    ]
    for cp in idx_cps:
        cp.start()
    zcps = [pltpu.make_async_copy(zeros_hbm, buf_v.at[b], sems[b])
            for b in range(NBUF)]
    for cp in zcps:
        cp.start()
    for cp in idx_cps:
        cp.wait()
    for cp in zcps:
        cp.wait()

    iota16 = lax.iota(jnp.int32, 16)
    zeros16 = jnp.zeros((16,), jnp.float32)
    ones16 = jnp.ones((16,), jnp.float32)

    def set_chunk(m, slot, vals16):
        jj = m // NCH
        c0 = (m % NCH) * CW
        slot16 = jnp.full((16,), slot, jnp.int32)
        for g in range(IW // 16):
            i16 = g * 16 + iota16
            cols = idx_v[pl.ds(jj * IW + g * 16, 16)]
            rel = cols - c0
            mask = (rel >= 0) & (rel < CW)
            plsc.store_scatter(buf_v, [slot16, rel, i16], vals16, mask=mask)

    def dma(slot, m):
        jj = j_base + m // NCH
        c0 = (m % NCH) * CW
        return pltpu.make_async_copy(
            buf_v.at[slot],
            out_hbm.at[jj, pl.ds(c0, CW), pl.ds(i_base, IW)],
            sems[slot],
        )

    for b in range(NBUF):
        set_chunk(b, b, ones16)
        dma(b, b).start()

    def step(t, carry):
        g = t * NBUF
        for b in range(NBUF):
            m = g + b
            dma(b, m).wait()
            set_chunk(m - NBUF, b, zeros16)
            set_chunk(m, b, ones16)
            dma(b, m).start()
        return carry

    lax.fori_loop(1, CHUNKS // NBUF, step, 0)

    for m in range(NBUF * (CHUNKS // NBUF), CHUNKS):
        b = m % NBUF
        dma(b, m - NBUF).wait()
        set_chunk(m - NBUF, b, zeros16)
        set_chunk(m, b, ones16)
        dma(b, m).start()

    for b in range(NBUF):
        dma(b, CHUNKS - NBUF + b).wait()


def kernel(x):
    xt = x.astype(jnp.int32).T.reshape(NJ * NI)
    zeros = jnp.zeros((CW, IW), jnp.float32)
    out = _onehot_sc(xt, zeros)
    return out.transpose(2, 0, 1)


# speedup vs baseline: 1.0246x; 1.0246x over previous
import functools

import jax
import jax.numpy as jnp
from jax import lax
from jax.experimental import pallas as pl
from jax.experimental.pallas import tpu as pltpu, tpu_sc as plsc

NUM_CLS = 1000
NI, NJ = 4096, 50
NIS = 16
IW = NI // NIS
JW = NJ // 2
CW = 200
NCH = NUM_CLS // CW
CHUNKS = JW * NCH
NBUF = 2

_mesh = plsc.VectorSubcoreMesh(core_axis_name="c", subcore_axis_name="s")


@functools.partial(
    pl.kernel,
    out_type=jax.ShapeDtypeStruct((NJ, NUM_CLS, NI), jnp.float32),
    mesh=_mesh,
    scratch_types=[
        pltpu.VMEM((JW * IW,), jnp.int32),
        pltpu.VMEM((NBUF, CW, IW), jnp.float32),
        pltpu.SemaphoreType.DMA,
        pltpu.SemaphoreType.DMA,
        pltpu.SemaphoreType.DMA,
    ],
    compiler_params=pltpu.CompilerParams(needs_layout_passes=False),
)
def _onehot_sc(xt_hbm, zeros_hbm, out_hbm, idx_v, buf_v, sem0, sem1, semz):
    wid = lax.axis_index("s") * 2 + lax.axis_index("c")
    i_base = (wid % NIS) * IW
    j_base = (wid // NIS) * JW
    sems = (sem0, sem1)

    idx_cps = [
        pltpu.make_async_copy(
            xt_hbm.at[pl.ds((j_base + jj) * NI + i_base, IW)],
            idx_v.at[pl.ds(jj * IW, IW)], semz)
        for jj in range(JW)
    ]
    for cp in idx_cps:
        cp.start()
    zcp = pltpu.make_async_copy(zeros_hbm, buf_v, sems[0])
    zcp.start()
    for cp in idx_cps:
        cp.wait()
    zcp.wait()

    iota16 = lax.iota(jnp.int32, 16)
    zeros16 = jnp.zeros((16,), jnp.float32)
    ones16 = jnp.ones((16,), jnp.float32)

    def set_chunk(m, slot, vals16):
        jj = m // NCH
        c0 = (m % NCH) * CW
        slot16 = jnp.full((16,), slot, jnp.int32)
        for g in range(IW // 16):
            i16 = g * 16 + iota16
            cols = idx_v[pl.ds(jj * IW + g * 16, 16)]
            rel = cols - c0
            mask = (rel >= 0) & (rel < CW)
            plsc.store_scatter(buf_v, [slot16, rel, i16], vals16, mask=mask)

    def dma(slot, m):
        jj = j_base + m // NCH
        c0 = (m % NCH) * CW
        return pltpu.make_async_copy(
            buf_v.at[slot],
            out_hbm.at[jj, pl.ds(c0, CW), pl.ds(i_base, IW)],
            sems[slot],
        )

    for b in range(NBUF):
        set_chunk(b, b, ones16)
        dma(b, b).start()

    def step(t, carry):
        g = t * NBUF
        for b in range(NBUF):
            m = g + b
            dma(b, m).wait()
            set_chunk(m - NBUF, b, zeros16)
            set_chunk(m, b, ones16)
            dma(b, m).start()
        return carry

    lax.fori_loop(1, CHUNKS // NBUF, step, 0)

    for m in range(NBUF * (CHUNKS // NBUF), CHUNKS):
        b = m % NBUF
        dma(b, m - NBUF).wait()
        set_chunk(m - NBUF, b, zeros16)
        set_chunk(m, b, ones16)
        dma(b, m).start()

    for b in range(NBUF):
        dma(b, CHUNKS - NBUF + b).wait()


def kernel(x):
    xt = x.astype(jnp.int32).T.reshape(NJ * NI)
    zeros = jnp.zeros((NBUF, CW, IW), jnp.float32)
    out = _onehot_sc(xt, zeros)
    return out.transpose(2, 0, 1)
